# Initial kernel scaffold; baseline (speedup 1.0000x reference)
import jax
import jax.numpy as jnp
from jax.experimental import pallas as pl


def kernel(x_user, x_item, edge_index_u2i, edge_index_i2u, edge_weight_u2i, edge_weight_i2u, batch_user, batch_item, W_msg_u2i, W_root_u2i, W_msg_i2u, W_root_i2u):
    raise NotImplementedError("write your pallas kernel here")


# speedup vs baseline: 5.9877x; 5.9877x over previous
import functools

import jax
import jax.numpy as jnp
from jax import lax
from jax.experimental import pallas as pl
from jax.experimental.pallas import tpu as pltpu
from jax.experimental.pallas import tpu_sc as plsc

N_USER = 10000
N_ITEM = 10000
D = 128
E = 320000

NS = 16
C = 128
B_BLK = 16
NBLK = 10
CHUNKS = B_BLK * NBLK
EPT = CHUNKS * C
E_PAD = NS * EPT
RPT = 632
RPT_LAST = N_USER - 15 * RPT

_f32 = jnp.float32


def _tc_body(xu_ref, xi_ref, wmu_ref, wru_ref, wmi_ref, wri_ref,
             yu_ref, ru_ref, yi_ref, ri_ref):
    xu = xu_ref[...]
    xi = xi_ref[...]
    yu_ref[...] = jnp.dot(xu, wmu_ref[...], preferred_element_type=_f32)
    ru_ref[...] = jnp.dot(xu, wri_ref[...], preferred_element_type=_f32)
    yi_ref[...] = jnp.dot(xi, wmi_ref[...], preferred_element_type=_f32)
    ri_ref[...] = jnp.dot(xi, wru_ref[...], preferred_element_type=_f32)


def _tc_transform(x_user, x_item, W_msg_u2i, W_root_u2i, W_msg_i2u, W_root_i2u):
    blk = 1000
    grid = N_USER // blk
    xspec = pl.BlockSpec((blk, D), lambda i: (i, 0))
    wspec = pl.BlockSpec((D, D), lambda i: (0, 0))
    out_sds = jax.ShapeDtypeStruct((N_USER, D), _f32)
    return pl.pallas_call(
        _tc_body,
        grid=(grid,),
        in_specs=[xspec, xspec, wspec, wspec, wspec, wspec],
        out_specs=[xspec, xspec, xspec, xspec],
        out_shape=[out_sds, out_sds, out_sds, out_sds],
    )(x_user, x_item, W_msg_u2i, W_root_u2i, W_msg_i2u, W_root_i2u)


def _sc_body(yu, yi, ru, ri,
             su2i, du2i, wu2i, si2u, di2u, wi2u,
             out_user, out_item,
             src_v, dst_v, w_v, rows_v, acc, sem):
    c = lax.axis_index("c")
    s = lax.axis_index("s")

    def copy_rows(src, dst):
        @pl.when(s < 15)
        def _():
            off = pl.multiple_of(s * RPT, 8)
            pltpu.sync_copy(src.at[pl.ds(off, RPT)], dst.at[pl.ds(off, RPT)])

        @pl.when(s == 15)
        def _():
            pltpu.sync_copy(src.at[pl.ds(15 * RPT, RPT_LAST)],
                            dst.at[pl.ds(15 * RPT, RPT_LAST)])

    def run_direction(y_hbm, r_hbm, src_hbm, dst_hbm, w_hbm, out_hbm):
        copy_rows(r_hbm, acc)
        plsc.subcore_barrier()

        def block_body(b, carry):
            blk = pl.ds(b * B_BLK, B_BLK)
            pltpu.sync_copy(src_hbm.at[s, blk], src_v)
            pltpu.sync_copy(dst_hbm.at[s, blk], dst_v)
            pltpu.sync_copy(w_hbm.at[s, blk], w_v)

            def chunk_body(k, carry2):
                pltpu.async_copy(y_hbm.at[src_v.at[k]], rows_v, sem).wait()

                def group_body(g, carry3):
                    w16 = w_v[k, pl.ds(g * 16, 16)]
                    for i in range(16):
                        ws = w16[i]
                        e = g * 16 + i
                        for j in range(D // 16):
                            sl = pl.ds(j * 16, 16)
                            rows_v[e, sl] = rows_v[e, sl] * ws
                    return carry3

                lax.fori_loop(0, C // 16, group_body, 0)

                pltpu.sync_copy(rows_v, acc.at[dst_v.at[k]], add=True)
                return carry2

            lax.fori_loop(0, B_BLK, chunk_body, 0)
            return carry

        lax.fori_loop(0, NBLK, block_body, 0)
        plsc.subcore_barrier()
        copy_rows(acc, out_hbm)

    @pl.when(c == 0)
    def _():
        run_direction(yu, ri, su2i, du2i, wu2i, out_item)

    @pl.when(c == 1)
    def _():
        run_direction(yi, ru, si2u, di2u, wi2u, out_user)


def _sc_conv(yu, yi, ru, ri, su2i, du2i, wu2i, si2u, di2u, wi2u):
    mesh = plsc.VectorSubcoreMesh(core_axis_name="c", subcore_axis_name="s")
    out_sds = jax.ShapeDtypeStruct((N_USER, D), _f32)
    kern = pl.kernel(
        _sc_body,
        out_type=(out_sds, out_sds),
        mesh=mesh,
        scratch_types=[
            pltpu.VMEM((B_BLK, C), jnp.int32),
            pltpu.VMEM((B_BLK, C), jnp.int32),
            pltpu.VMEM((B_BLK, C), _f32),
            pltpu.VMEM((C, D), _f32),
            pltpu.VMEM_SHARED((N_USER, D), _f32),
            pltpu.SemaphoreType.DMA,
        ],
    )
    return kern(yu, yi, ru, ri, su2i, du2i, wu2i, si2u, di2u, wi2u)


def _pad_edges(edge_index, w, n_src, n_dst):
    pad = E_PAD - E
    src = edge_index[0].astype(jnp.int32)
    dst = edge_index[1].astype(jnp.int32)
    ar = jnp.arange(pad, dtype=jnp.int32)
    src_p = jnp.concatenate([src, ar % n_src]).reshape(NS, CHUNKS, C)
    dst_p = jnp.concatenate([dst, ar % n_dst]).reshape(NS, CHUNKS, C)
    w_p = jnp.concatenate([w, jnp.zeros((pad,), _f32)]).reshape(NS, CHUNKS, C)
    return src_p, dst_p, w_p


def kernel(x_user, x_item, edge_index_u2i, edge_index_i2u,
           edge_weight_u2i, edge_weight_i2u, batch_user, batch_item,
           W_msg_u2i, W_root_u2i, W_msg_i2u, W_root_i2u):
    yu, ru, yi, ri = _tc_transform(x_user, x_item, W_msg_u2i, W_root_u2i,
                                   W_msg_i2u, W_root_i2u)
    su2i, du2i, wu2i = _pad_edges(edge_index_u2i, edge_weight_u2i,
                                  N_USER, N_ITEM)
    si2u, di2u, wi2u = _pad_edges(edge_index_i2u, edge_weight_i2u,
                                  N_ITEM, N_USER)
    out_user, out_item = _sc_conv(yu, yi, ru, ri,
                                  su2i, du2i, wu2i, si2u, di2u, wi2u)
    return (out_user, out_item)
